# baseline (speedup 1.0000x reference)
import functools

import jax
import jax.numpy as jnp
from jax import lax
from jax.experimental import pallas as pl
from jax.experimental.pallas import tpu as pltpu
from jax.experimental.pallas import tpu_sc as plsc

B, N, C = 16, 128, 64
NUM_CORES = 2
NUM_SUBCORES = 16
NW = NUM_CORES * NUM_SUBCORES
ROWS_PER_W = (B * N) // NW
CB = 64
V2 = 50000


def _sc_gather2(emb2, idx_flat):
    mesh = plsc.VectorSubcoreMesh(core_axis_name="c", subcore_axis_name="s")

    @functools.partial(
        pl.kernel,
        mesh=mesh,
        out_type=jax.ShapeDtypeStruct((B * N, 2 * C), jnp.float32),
        scratch_types=[
            pltpu.VMEM((ROWS_PER_W,), jnp.int32),
            pltpu.VMEM((ROWS_PER_W, 2 * C), jnp.float32),
            pltpu.SemaphoreType.DMA,
            pltpu.SemaphoreType.DMA,
        ],
    )
    def gather_kernel(table_hbm, idx_hbm, out_hbm, idx_v, rows_v, sem_i, sem_r):
        wid = lax.axis_index("s") * NUM_CORES + lax.axis_index("c")
        base = wid * ROWS_PER_W
        pltpu.async_copy(idx_hbm.at[pl.ds(base, ROWS_PER_W)], idx_v, sem_i).wait()
        copies = []
        for k in range(ROWS_PER_W // 16):
            chunk = idx_v[pl.ds(k * 16, 16)] >> 1
            for l in range(16):
                r = k * 16 + l
                c = pltpu.make_async_copy(
                    table_hbm.at[pl.ds(chunk[l], 1)],
                    rows_v.at[pl.ds(r, 1)],
                    sem_r,
                )
                c.start()
                copies.append(c)
        for c in copies:
            c.wait()
        pltpu.sync_copy(rows_v, out_hbm.at[pl.ds(base, ROWS_PER_W)])

    return gather_kernel(emb2, idx_flat)


def _tc_diag_body(nh_ref, idxn_ref, out_ref):
    h2 = nh_ref[0]
    idx_n = idxn_ref[0]
    odd = (idx_n & 1) == 1
    h = jnp.where(odd, h2[:, C:], h2[:, :C])
    hm = jnp.where(idx_n != 0, h, 0.0)
    ht = hm.T
    i = lax.broadcasted_iota(jnp.int32, (CB, N, N), 1)
    j = lax.broadcasted_iota(jnp.int32, (CB, N, N), 2)
    out_ref[0] = jnp.where(i == j, ht[:, None, :], 0.0)


def _tc_diag(node_h2, idx_n):
    return pl.pallas_call(
        _tc_diag_body,
        grid=(B, C // CB),
        in_specs=[
            pl.BlockSpec((1, N, 2 * C), lambda b, cb: (b, 0, 0)),
            pl.BlockSpec((1, N, 1), lambda b, cb: (b, 0, 0)),
        ],
        out_specs=pl.BlockSpec((1, CB, N, N), lambda b, cb: (b, cb, 0, 0)),
        out_shape=jax.ShapeDtypeStruct((B, C, N, N), jnp.float32),
    )(node_h2, idx_n)


def kernel(batch_node_attr, emb):
    idx = batch_node_attr[:, :, 0].astype(jnp.int32)
    emb2 = emb[:V2 * 2].reshape(V2, 2 * C)
    node_h2 = _sc_gather2(emb2, idx.reshape(-1))
    return _tc_diag(node_h2.reshape(B, N, 2 * C), idx.reshape(B, N, 1))


# speedup vs baseline: 1.0142x; 1.0142x over previous
import functools

import jax
import jax.numpy as jnp
from jax import lax
from jax.experimental import pallas as pl
from jax.experimental.pallas import tpu as pltpu
from jax.experimental.pallas import tpu_sc as plsc

B, N, C = 16, 128, 4 * 16
NUM_CORES = 2
NUM_SUBCORES = 16
NW = NUM_CORES * NUM_SUBCORES
ROWS_PER_W = (B * N) // NW
CB = 64


def _sc_gather(emb, idx_flat):
    mesh = plsc.VectorSubcoreMesh(core_axis_name="c", subcore_axis_name="s")

    @functools.partial(
        pl.kernel,
        mesh=mesh,
        out_type=jax.ShapeDtypeStruct((B * N, C), jnp.float32),
        scratch_types=[
            pltpu.VMEM((ROWS_PER_W,), jnp.int32),
            pltpu.VMEM((ROWS_PER_W, C), jnp.float32),
            pltpu.SemaphoreType.DMA,
        ],
        compiler_params=pltpu.CompilerParams(use_tc_tiling_on_sc=False),
    )
    def gather_kernel(table_hbm, idx_hbm, out_hbm, idx_v, rows_v, sem):
        wid = lax.axis_index("s") * NUM_CORES + lax.axis_index("c")
        base = wid * ROWS_PER_W
        pltpu.sync_copy(idx_hbm.at[pl.ds(base, ROWS_PER_W)], idx_v)
        pltpu.async_copy(table_hbm.at[idx_v], rows_v, sem).wait()
        pltpu.sync_copy(rows_v, out_hbm.at[pl.ds(base, ROWS_PER_W)])

    return gather_kernel(emb, idx_flat)


def _tc_diag_body(nh_ref, idx_ref, out_ref):
    h = nh_ref[0]
    m = idx_ref[0] != 0
    ht = jnp.where(m, h.T, 0.0)
    i = lax.broadcasted_iota(jnp.int32, (CB, N, N), 1)
    j = lax.broadcasted_iota(jnp.int32, (CB, N, N), 2)
    out_ref[0] = jnp.where(i == j, ht[:, None, :], 0.0)


def _tc_diag(node_h, idx3):
    return pl.pallas_call(
        _tc_diag_body,
        grid=(B, C // CB),
        in_specs=[
            pl.BlockSpec((1, N, C), lambda b, cb: (b, 0, 0)),
            pl.BlockSpec((1, 1, N), lambda b, cb: (b, 0, 0)),
        ],
        out_specs=pl.BlockSpec((1, CB, N, N), lambda b, cb: (b, cb, 0, 0)),
        out_shape=jax.ShapeDtypeStruct((B, C, N, N), jnp.float32),
    )(node_h, idx3)


def kernel(batch_node_attr, emb):
    idx = batch_node_attr[:, :, 0].astype(jnp.int32)
    node_h = _sc_gather(emb, idx.reshape(-1))
    return _tc_diag(node_h.reshape(B, N, C), idx.reshape(B, 1, N))


# speedup vs baseline: 1.1825x; 1.1660x over previous
import functools

import jax
import jax.numpy as jnp
from jax import lax
from jax.experimental import pallas as pl
from jax.experimental.pallas import tpu as pltpu
from jax.experimental.pallas import tpu_sc as plsc

B, N, C = 16, 128, 64
V = 100001
NUM_CORES = 2
NUM_SUBCORES = 16
NW = NUM_CORES * NUM_SUBCORES
ROWS_PER_W = (B * N) // NW
CB = 64
TCOL = 2048


def _tc_transpose_body(et_ref, out_ref):
    out_ref[...] = et_ref[...].T


def _tc_transpose(emb_t):
    steps = (V + TCOL - 1) // TCOL
    return pl.pallas_call(
        _tc_transpose_body,
        grid=(steps,),
        in_specs=[pl.BlockSpec((C, TCOL), lambda g: (0, g))],
        out_specs=pl.BlockSpec((TCOL, C), lambda g: (g, 0)),
        out_shape=jax.ShapeDtypeStruct((V, C), jnp.float32),
    )(emb_t)


def _sc_gather(table, idx_flat):
    mesh = plsc.VectorSubcoreMesh(core_axis_name="c", subcore_axis_name="s")

    @functools.partial(
        pl.kernel,
        mesh=mesh,
        out_type=jax.ShapeDtypeStruct((B * N, C), jnp.float32),
        scratch_types=[
            pltpu.VMEM((ROWS_PER_W,), jnp.int32),
            pltpu.VMEM((ROWS_PER_W, C), jnp.float32),
            pltpu.SemaphoreType.DMA,
            pltpu.SemaphoreType.DMA,
        ],
    )
    def gather_kernel(table_hbm, idx_hbm, out_hbm, idx_v, rows_v, sem_i, sem_r):
        wid = lax.axis_index("s") * NUM_CORES + lax.axis_index("c")
        base = wid * ROWS_PER_W
        pltpu.async_copy(idx_hbm.at[pl.ds(base, ROWS_PER_W)], idx_v, sem_i).wait()
        copies = []
        for k in range(ROWS_PER_W // 16):
            chunk = idx_v[pl.ds(k * 16, 16)]
            for l in range(16):
                r = k * 16 + l
                c = pltpu.make_async_copy(
                    table_hbm.at[pl.ds(chunk[l], 1)],
                    rows_v.at[pl.ds(r, 1)],
                    sem_r,
                )
                c.start()
                copies.append(c)
        for c in copies:
            c.wait()
        pltpu.sync_copy(rows_v, out_hbm.at[pl.ds(base, ROWS_PER_W)])

    return gather_kernel(table, idx_flat)


def _tc_diag_body(nh_ref, idx_ref, out_ref):
    h = nh_ref[0]
    m = idx_ref[0] != 0
    ht = jnp.where(m, h.T, 0.0)
    i = lax.broadcasted_iota(jnp.int32, (CB, N, N), 1)
    j = lax.broadcasted_iota(jnp.int32, (CB, N, N), 2)
    out_ref[0] = jnp.where(i == j, ht[:, None, :], 0.0)


def _tc_diag(node_h, idx3):
    return pl.pallas_call(
        _tc_diag_body,
        grid=(B, C // CB),
        in_specs=[
            pl.BlockSpec((1, N, C), lambda b, cb: (b, 0, 0)),
            pl.BlockSpec((1, 1, N), lambda b, cb: (b, 0, 0)),
        ],
        out_specs=pl.BlockSpec((1, CB, N, N), lambda b, cb: (b, cb, 0, 0)),
        out_shape=jax.ShapeDtypeStruct((B, C, N, N), jnp.float32),
    )(node_h, idx3)


def kernel(batch_node_attr, emb):
    idx = batch_node_attr[:, :, 0].astype(jnp.int32)
    table = _tc_transpose(emb.T)
    node_h = _sc_gather(table, idx.reshape(-1))
    return _tc_diag(node_h.reshape(B, N, C), idx.reshape(B, 1, N))


# speedup vs baseline: 1.3860x; 1.1720x over previous
import functools

import jax
import jax.numpy as jnp
from jax import lax
from jax.experimental import pallas as pl
from jax.experimental.pallas import tpu as pltpu
from jax.experimental.pallas import tpu_sc as plsc

B, N, C = 16, 128, 64
V = 100001
G = 12544
S = 8
GC = S * C
TCOL = 1792
TSTEPS = G // TCOL
NUM_CORES = 2
NUM_SUBCORES = 16
NW = NUM_CORES * NUM_SUBCORES
ROWS_PER_W = (B * N) // NW
CB = 64


def _tc_group_body(*refs):
    in_refs, out_ref = refs[:S], refs[S]
    out_ref[...] = jnp.concatenate([r[...].T for r in in_refs], axis=1)


def _tc_group(emb_t):
    return pl.pallas_call(
        _tc_group_body,
        grid=(TSTEPS,),
        in_specs=[
            pl.BlockSpec((C, TCOL), functools.partial(lambda s, t: (0, t + s * TSTEPS), s))
            for s in range(S)
        ],
        out_specs=pl.BlockSpec((TCOL, GC), lambda t: (t, 0)),
        out_shape=jax.ShapeDtypeStruct((G, GC), jnp.float32),
    )(*([emb_t] * S))


def _sc_gather(table_g, idx_flat):
    mesh = plsc.VectorSubcoreMesh(core_axis_name="c", subcore_axis_name="s")

    @functools.partial(
        pl.kernel,
        mesh=mesh,
        out_type=jax.ShapeDtypeStruct((B * N, GC), jnp.float32),
        scratch_types=[
            pltpu.VMEM((ROWS_PER_W,), jnp.int32),
            pltpu.VMEM((ROWS_PER_W, GC), jnp.float32),
            pltpu.SemaphoreType.DMA,
            pltpu.SemaphoreType.DMA,
        ],
    )
    def gather_kernel(table_hbm, idx_hbm, out_hbm, idx_v, rows_v, sem_i, sem_r):
        wid = lax.axis_index("s") * NUM_CORES + lax.axis_index("c")
        base = wid * ROWS_PER_W
        pltpu.async_copy(idx_hbm.at[pl.ds(base, ROWS_PER_W)], idx_v, sem_i).wait()
        copies = []
        for k in range(ROWS_PER_W // 16):
            chunk = lax.rem(idx_v[pl.ds(k * 16, 16)], G)
            for l in range(16):
                r = k * 16 + l
                c = pltpu.make_async_copy(
                    table_hbm.at[pl.ds(chunk[l], 1)],
                    rows_v.at[pl.ds(r, 1)],
                    sem_r,
                )
                c.start()
                copies.append(c)
        for c in copies:
            c.wait()
        pltpu.sync_copy(rows_v, out_hbm.at[pl.ds(base, ROWS_PER_W)])

    return gather_kernel(table_g, idx_flat)


def _tc_diag_body(nh_ref, idxn_ref, out_ref):
    h8 = nh_ref[0]
    idx_n = idxn_ref[0]
    s_n = idx_n // G
    h = h8[:, :C]
    for s in range(1, S):
        h = jnp.where(s_n == s, h8[:, s * C:(s + 1) * C], h)
    hm = jnp.where(idx_n != 0, h, 0.0)
    ht = hm.T
    i = lax.broadcasted_iota(jnp.int32, (CB, N, N), 1)
    j = lax.broadcasted_iota(jnp.int32, (CB, N, N), 2)
    out_ref[0] = jnp.where(i == j, ht[:, None, :], 0.0)


def _tc_diag(node_h8, idx_n):
    return pl.pallas_call(
        _tc_diag_body,
        grid=(B, C // CB),
        in_specs=[
            pl.BlockSpec((1, N, GC), lambda b, cb: (b, 0, 0)),
            pl.BlockSpec((1, N, 1), lambda b, cb: (b, 0, 0)),
        ],
        out_specs=pl.BlockSpec((1, CB, N, N), lambda b, cb: (b, cb, 0, 0)),
        out_shape=jax.ShapeDtypeStruct((B, C, N, N), jnp.float32),
    )(node_h8, idx_n)


def kernel(batch_node_attr, emb):
    idx = batch_node_attr[:, :, 0].astype(jnp.int32)
    table_g = _tc_group(emb.T)
    node_h8 = _sc_gather(table_g, idx.reshape(-1))
    return _tc_diag(node_h8.reshape(B, N, GC), idx.reshape(B, N, 1))


# speedup vs baseline: 1.3891x; 1.0023x over previous
import functools

import jax
import jax.numpy as jnp
from jax import lax
from jax.experimental import pallas as pl
from jax.experimental.pallas import tpu as pltpu
from jax.experimental.pallas import tpu_sc as plsc

B, N, C = 16, 128, 64
V = 100001
G = 12544
S = 8
GC = S * C
TCOL = 1792
TSTEPS = G // TCOL
NUM_CORES = 2
NUM_SUBCORES = 16
NW = NUM_CORES * NUM_SUBCORES
ROWS_PER_W = (B * N) // NW
CB = 64


def _tc_group_body(*refs):
    in_refs, out_ref = refs[:S], refs[S]
    out_ref[...] = jnp.concatenate([r[...].T for r in in_refs], axis=1)


def _tc_group(emb_t):
    return pl.pallas_call(
        _tc_group_body,
        grid=(TSTEPS,),
        in_specs=[
            pl.BlockSpec((C, TCOL), functools.partial(lambda s, t: (0, t + s * TSTEPS), s))
            for s in range(S)
        ],
        out_specs=pl.BlockSpec((TCOL, GC), lambda t: (t, 0)),
        out_shape=jax.ShapeDtypeStruct((G, GC), jnp.float32),
    )(*([emb_t] * S))


def _sc_gather(table_g, idx_flat):
    mesh = plsc.VectorSubcoreMesh(core_axis_name="c", subcore_axis_name="s")

    @functools.partial(
        pl.kernel,
        mesh=mesh,
        out_type=jax.ShapeDtypeStruct((B, N, GC), jnp.float32),
        scratch_types=[
            pltpu.VMEM((ROWS_PER_W,), jnp.int32),
            pltpu.VMEM((1, ROWS_PER_W, GC), jnp.float32),
            pltpu.SemaphoreType.DMA,
            pltpu.SemaphoreType.DMA,
        ],
    )
    def gather_kernel(table_hbm, idx_hbm, out_hbm, idx_v, rows_v, sem_i, sem_r):
        wid = lax.axis_index("s") * NUM_CORES + lax.axis_index("c")
        base = wid * ROWS_PER_W
        pltpu.async_copy(idx_hbm.at[pl.ds(base, ROWS_PER_W)], idx_v, sem_i).wait()
        copies = []
        for k in range(ROWS_PER_W // 16):
            chunk = lax.rem(idx_v[pl.ds(k * 16, 16)], G)
            for l in range(16):
                r = k * 16 + l
                c = pltpu.make_async_copy(
                    table_hbm.at[pl.ds(chunk[l], 1)],
                    rows_v.at[0, pl.ds(r, 1)],
                    sem_r,
                )
                c.start()
                copies.append(c)
        for c in copies:
            c.wait()
        b0 = base // N
        r0 = base - b0 * N
        pltpu.sync_copy(rows_v, out_hbm.at[pl.ds(b0, 1), pl.ds(r0, ROWS_PER_W)])

    return gather_kernel(table_g, idx_flat)


def _tc_diag_body(nh_ref, idxn_ref, out_ref):
    h8 = nh_ref[0]
    idx_n = idxn_ref[0]
    s_n = idx_n // G
    h = h8[:, :C]
    for s in range(1, S):
        h = jnp.where(s_n == s, h8[:, s * C:(s + 1) * C], h)
    hm = jnp.where(idx_n != 0, h, 0.0)
    ht = hm.T
    i = lax.broadcasted_iota(jnp.int32, (CB, N, N), 1)
    j = lax.broadcasted_iota(jnp.int32, (CB, N, N), 2)
    out_ref[0] = jnp.where(i == j, ht[:, None, :], 0.0)


def _tc_diag(node_h8, idx_n):
    return pl.pallas_call(
        _tc_diag_body,
        grid=(B, C // CB),
        in_specs=[
            pl.BlockSpec((1, N, GC), lambda b, cb: (b, 0, 0)),
            pl.BlockSpec((1, N, 1), lambda b, cb: (b, 0, 0)),
        ],
        out_specs=pl.BlockSpec((1, CB, N, N), lambda b, cb: (b, cb, 0, 0)),
        out_shape=jax.ShapeDtypeStruct((B, C, N, N), jnp.float32),
    )(node_h8, idx_n)


def kernel(batch_node_attr, emb):
    idx = batch_node_attr[:, :, 0].astype(jnp.int32)
    table_g = _tc_group(emb.T)
    node_h8 = _sc_gather(table_g, idx.reshape(-1))
    return _tc_diag(node_h8, idx.reshape(B, N, 1))
